# baseline (speedup 1.0000x reference)
import jax
import jax.numpy as jnp
from jax.experimental import pallas as pl
from jax.experimental.pallas import tpu as pltpu

VOCAB = 32000
PAD = 0
SMOOTH = 0.1
CONF = 1.0 - SMOOTH
EPS = SMOOTH / (VOCAB - 2)

ROW_BLK = 128


def _loss_kernel(tgt_ref, pred_ref, out_ref, lse_ref):
    x = pred_ref[...]
    tloc = tgt_ref[0]
    lane = jax.lax.broadcasted_iota(jnp.int32, (ROW_BLK, VOCAB), 1)
    blk_pt = jnp.sum(jnp.where(lane == tloc, x, 0.0), axis=1, keepdims=True)
    blk_tot = jnp.sum(x, axis=1, keepdims=True)
    s_raw = jnp.sum(jnp.exp(x), axis=1, keepdims=True)
    lse_ref[...] = jnp.log(s_raw)
    bad = jnp.logical_or(~jnp.isfinite(s_raw), s_raw == 0.0)

    @pl.when(jnp.any(bad))
    def _stable():
        m = jnp.max(x, axis=1, keepdims=True)
        s = jnp.sum(jnp.exp(x - m), axis=1, keepdims=True)
        lse_ref[...] = jnp.where(bad, m + jnp.log(s), lse_ref[...])

    lse = lse_ref[...]
    s_row = blk_tot - VOCAB * lse
    lp0 = x[:, 0:1] - lse
    lpt = blk_pt - lse
    c0 = SMOOTH * jnp.log(EPS) + CONF * jnp.log(CONF)
    row_loss = c0 - EPS * (s_row - lp0 - lpt) - CONF * lpt
    row_loss = jnp.where(tloc != PAD, row_loss, 0.0)
    out_ref[...] = jnp.sum(row_loss).reshape(1, 1, 1)


@jax.jit
def kernel(pred, target):
    n, v = pred.shape
    n_i = n // ROW_BLK
    tgt3 = target.astype(jnp.int32).reshape(n_i, ROW_BLK, 1)
    parts = pl.pallas_call(
        _loss_kernel,
        grid=(n_i,),
        in_specs=[
            pl.BlockSpec((1, ROW_BLK, 1), lambda i: (i, 0, 0)),
            pl.BlockSpec((ROW_BLK, v), lambda i: (i, 0)),
        ],
        out_specs=pl.BlockSpec((1, 1, 1), lambda i: (i, 0, 0)),
        out_shape=jax.ShapeDtypeStruct((n_i, 1, 1), jnp.float32),
        scratch_shapes=[pltpu.VMEM((ROW_BLK, 1), jnp.float32)],
        compiler_params=pltpu.CompilerParams(
            dimension_semantics=("parallel",)),
    )(tgt3, pred)
    return jnp.sum(parts)
